# Initial kernel scaffold; baseline (speedup 1.0000x reference)
import jax
import jax.numpy as jnp
from jax.experimental import pallas as pl


def kernel(node_feats_r, edge_feats_r, node_feats_p, edge_feats_p, Wn, bn, We, be, Wa, ba, Wb, bb, edge_index_r, seg_r, edge_index_p, seg_p):
    raise NotImplementedError("write your pallas kernel here")


# speedup vs baseline: 3.1149x; 3.1149x over previous
import functools

import jax
import jax.numpy as jnp
from jax import lax
from jax.experimental import pallas as pl
from jax.experimental.pallas import tpu as pltpu
from jax.experimental.pallas import tpu_sc as plsc

N = 4096
E = 16384
B = 16
D = 300
DP = 384
DEPTH = 3
NC = 2
NS = 16
EC = 128
TPE = E // NS
NCHUNK = TPE // EC
RPT = N // NS
LANES = 16


def _pad2(w, rows, cols):
    return jnp.zeros((rows, cols), w.dtype).at[: w.shape[0], : w.shape[1]].set(w)


def _linear(x, w, b, relu, block_rows):
    m, k = x.shape
    dp = w.shape[1]

    def body(x_ref, w_ref, b_ref, o_ref):
        y = jnp.dot(x_ref[...], w_ref[...],
                    preferred_element_type=jnp.float32) + b_ref[...]
        o_ref[...] = jnp.maximum(y, 0.0) if relu else y

    return pl.pallas_call(
        body,
        grid=(m // block_rows,),
        in_specs=[
            pl.BlockSpec((block_rows, k), lambda i: (i, 0)),
            pl.BlockSpec((k, dp), lambda i: (0, 0)),
            pl.BlockSpec((1, dp), lambda i: (0, 0)),
        ],
        out_specs=pl.BlockSpec((block_rows, dp), lambda i: (i, 0)),
        out_shape=jax.ShapeDtypeStruct((m, dp), jnp.float32),
    )(x, w, b)


def _mlp(h, agg, wa, ba, wb, bb, relu, block_rows=512):
    def body(h_ref, a_ref, wa_ref, ba_ref, wb_ref, bb_ref, o_ref):
        z = h_ref[...] + a_ref[...]
        t = jnp.maximum(
            jnp.dot(z, wa_ref[...], preferred_element_type=jnp.float32)
            + ba_ref[...], 0.0)
        y = jnp.dot(t, wb_ref[...],
                    preferred_element_type=jnp.float32) + bb_ref[...]
        o_ref[...] = jnp.maximum(y, 0.0) if relu else y

    return pl.pallas_call(
        body,
        grid=(N // block_rows,),
        in_specs=[
            pl.BlockSpec((block_rows, DP), lambda i: (i, 0)),
            pl.BlockSpec((block_rows, DP), lambda i: (i, 0)),
            pl.BlockSpec((DP, DP), lambda i: (0, 0)),
            pl.BlockSpec((1, DP), lambda i: (0, 0)),
            pl.BlockSpec((DP, DP), lambda i: (0, 0)),
            pl.BlockSpec((1, DP), lambda i: (0, 0)),
        ],
        out_specs=pl.BlockSpec((block_rows, DP), lambda i: (i, 0)),
        out_shape=jax.ShapeDtypeStruct((N, DP), jnp.float32),
    )(h, agg, wa, ba, wb, bb)


def _final(h_r, agg_r, h_p, agg_p, wa, ba, wb, bb, seg_r3, seg_p3,
           block_rows=512):
    nblk = N // block_rows

    def body(hr, ar, hp, ap, wa_ref, ba_ref, wb_ref, bb_ref, sr, sp,
             diff_ref, re_ref, pr_ref):
        j = pl.program_id(0)

        def head(h_ref, a_ref):
            z = h_ref[...] + a_ref[...]
            t = jnp.maximum(
                jnp.dot(z, wa_ref[...], preferred_element_type=jnp.float32)
                + ba_ref[...], 0.0)
            return jnp.dot(t, wb_ref[...],
                           preferred_element_type=jnp.float32) + bb_ref[...]

        o_r = head(hr, ar)
        o_p = head(hp, ap)
        iota = lax.broadcasted_iota(jnp.int32, (B, block_rows), 0)
        ct_r = jnp.dot((sr[0] == iota).astype(jnp.float32), o_r,
                       preferred_element_type=jnp.float32)
        ct_p = jnp.dot((sp[0] == iota).astype(jnp.float32), o_p,
                       preferred_element_type=jnp.float32)

        @pl.when(j == 0)
        def _():
            re_ref[...] = ct_r
            pr_ref[...] = ct_p

        @pl.when(j > 0)
        def _():
            re_ref[...] += ct_r
            pr_ref[...] += ct_p

        @pl.when(j == nblk - 1)
        def _():
            diff_ref[...] = re_ref[...] - pr_ref[...]

    row_spec = pl.BlockSpec((block_rows, DP), lambda i: (i, 0))
    full = pl.BlockSpec((DP, DP), lambda i: (0, 0))
    bias = pl.BlockSpec((1, DP), lambda i: (0, 0))
    seg_spec = pl.BlockSpec((1, 1, block_rows), lambda i: (i, 0, 0))
    out_spec = pl.BlockSpec((B, DP), lambda i: (0, 0))
    return pl.pallas_call(
        body,
        grid=(nblk,),
        in_specs=[row_spec, row_spec, row_spec, row_spec, full, bias, full,
                  bias, seg_spec, seg_spec],
        out_specs=(out_spec, out_spec, out_spec),
        out_shape=(jax.ShapeDtypeStruct((B, DP), jnp.float32),
                   jax.ShapeDtypeStruct((B, DP), jnp.float32),
                   jax.ShapeDtypeStruct((B, DP), jnp.float32)),
    )(h_r, agg_r, h_p, agg_p, wa, ba, wb, bb, seg_r3, seg_p3)


def _edge_agg(h_r, e_r, src_r, dst_r, h_p, e_p, src_p, dst_p):
    mesh = plsc.VectorSubcoreMesh(core_axis_name="c", subcore_axis_name="s",
                                  num_cores=NC, num_subcores=NS)

    @functools.partial(
        pl.kernel,
        out_type=(jax.ShapeDtypeStruct((N, DP), jnp.float32),
                  jax.ShapeDtypeStruct((N, DP), jnp.float32)),
        mesh=mesh,
        scratch_types=[
            pltpu.VMEM((EC,), jnp.int32),
            pltpu.VMEM((EC,), jnp.int32),
            pltpu.VMEM((EC, DP), jnp.float32),
            pltpu.VMEM((EC, DP), jnp.float32),
            pltpu.SemaphoreType.DMA,
        ],
    )
    def k(hr, er, sr, dr, hp, ep, sp, dpp, out_r, out_p,
          sidx, didx, hbuf, ebuf, sem):
        cid = lax.axis_index("c")
        sid = lax.axis_index("s")

        def zrow(r, carry):
            for j in range(DP // LANES):
                hbuf[r, pl.ds(j * LANES, LANES)] = jnp.zeros((LANES,),
                                                             jnp.float32)
            return carry

        lax.fori_loop(0, EC, zrow, 0)
        row0 = sid * RPT

        @pl.when(cid == 0)
        def _():
            for q in range(RPT // EC):
                pltpu.sync_copy(hbuf, out_r.at[pl.ds(row0 + q * EC, EC)])

        @pl.when(cid == 1)
        def _():
            for q in range(RPT // EC):
                pltpu.sync_copy(hbuf, out_p.at[pl.ds(row0 + q * EC, EC)])

        plsc.subcore_barrier()

        def work(h, e, src, dst, out):
            base0 = sid * TPE

            def chunk(kk, carry):
                base = base0 + kk * EC
                pltpu.sync_copy(src.at[pl.ds(base, EC)], sidx)
                pltpu.sync_copy(dst.at[pl.ds(base, EC)], didx)
                pltpu.async_copy(h.at[sidx], hbuf, sem).wait()
                pltpu.sync_copy(e.at[pl.ds(base, EC)], ebuf)

                def row(r, c2):
                    for j in range(DP // LANES):
                        sl = pl.ds(j * LANES, LANES)
                        hbuf[r, sl] = jnp.maximum(hbuf[r, sl] + ebuf[r, sl],
                                                  0.0)
                    return c2

                lax.fori_loop(0, EC, row, 0)
                pltpu.sync_copy(hbuf, out.at[didx], add=True)
                return carry

            lax.fori_loop(0, NCHUNK, chunk, 0)

        @pl.when(cid == 0)
        def _():
            work(hr, er, sr, dr, out_r)

        @pl.when(cid == 1)
        def _():
            work(hp, ep, sp, dpp, out_p)

    return k(h_r, e_r, src_r, dst_r, h_p, e_p, src_p, dst_p)


def kernel(node_feats_r, edge_feats_r, node_feats_p, edge_feats_p,
           Wn, bn, We, be, Wa, ba, Wb, bb,
           edge_index_r, seg_r, edge_index_p, seg_p):
    f32 = jnp.float32
    wn = _pad2(Wn, 64, DP)
    we = _pad2(We, 8, DP)
    bn2 = _pad2(bn[None, :], 1, DP)
    be2 = _pad2(be[None, :], 1, DP)
    wa = [_pad2(Wa[i], DP, DP) for i in range(DEPTH)]
    wb = [_pad2(Wb[i], DP, DP) for i in range(DEPTH)]
    ba2 = [_pad2(ba[i][None, :], 1, DP) for i in range(DEPTH)]
    bb2 = [_pad2(bb[i][None, :], 1, DP) for i in range(DEPTH)]

    src_r = edge_index_r[0].astype(jnp.int32)
    dst_r = edge_index_r[1].astype(jnp.int32)
    src_p = edge_index_p[0].astype(jnp.int32)
    dst_p = edge_index_p[1].astype(jnp.int32)
    seg_r3 = seg_r.astype(jnp.int32).reshape(N // 512, 1, 512)
    seg_p3 = seg_p.astype(jnp.int32).reshape(N // 512, 1, 512)

    h_r = _linear(node_feats_r.astype(f32), wn, bn2, True, 512)
    h_p = _linear(node_feats_p.astype(f32), wn, bn2, True, 512)
    e_r = _linear(edge_feats_r.astype(f32), we, be2, False, 2048)
    e_p = _linear(edge_feats_p.astype(f32), we, be2, False, 2048)

    for i in range(DEPTH - 1):
        agg_r, agg_p = _edge_agg(h_r, e_r, src_r, dst_r, h_p, e_p, src_p,
                                 dst_p)
        h_r = _mlp(h_r, agg_r, wa[i], ba2[i], wb[i], bb2[i], True)
        h_p = _mlp(h_p, agg_p, wa[i], ba2[i], wb[i], bb2[i], True)

    agg_r, agg_p = _edge_agg(h_r, e_r, src_r, dst_r, h_p, e_p, src_p, dst_p)
    diff, react, prod = _final(h_r, agg_r, h_p, agg_p, wa[2], ba2[2], wb[2],
                               bb2[2], seg_r3, seg_p3)
    return (diff[:, :D], react[:, :D], prod[:, :D])
